# baseline (speedup 1.0000x reference)
import functools

import jax
import jax.numpy as jnp
from jax import lax
from jax.experimental import pallas as pl
from jax.experimental.pallas import tpu as pltpu
from jax.experimental.pallas import tpu_sc as plsc

_LANES = 16
_CHUNK = 128


@functools.lru_cache(maxsize=None)
def _make_gather(total_rows, feat, rows_per_batch, table_rows_per_batch):
    info = plsc.get_sparse_core_info()
    nc, ns = info.num_cores, info.num_subcores
    nw = nc * ns
    per_w = total_rows // nw
    n_chunks = per_w // _CHUNK
    chunk_elems = _CHUNK * feat
    table_elems = table_rows_per_batch * feat
    assert n_chunks % 2 == 0
    assert rows_per_batch % per_w == 0
    mesh = plsc.VectorSubcoreMesh(core_axis_name="c", subcore_axis_name="s")

    @functools.partial(
        pl.kernel,
        mesh=mesh,
        compiler_params=pltpu.CompilerParams(use_tc_tiling_on_sc=False,
                                             needs_layout_passes=False),
        out_type=jax.ShapeDtypeStruct((total_rows * feat,), jnp.float32),
        scratch_types=[
            pltpu.VMEM((per_w,), jnp.int32),
            pltpu.VMEM((table_elems,), jnp.float32),
            pltpu.VMEM((chunk_elems,), jnp.float32),
            pltpu.VMEM((chunk_elems,), jnp.float32),
            pltpu.SemaphoreType.DMA,
            pltpu.SemaphoreType.DMA,
        ],
    )
    def gather_kernel(idx_hbm, table_hbm, out_hbm, idx_v, table_v,
                      rows0, rows1, ss0, ss1):
        wid = lax.axis_index("s") * nc + lax.axis_index("c")
        base = wid * per_w
        b = base // rows_per_batch

        pltpu.sync_copy(idx_hbm.at[pl.ds(base, per_w)], idx_v)
        pltpu.sync_copy(table_hbm.at[pl.ds(b * table_elems, table_elems)],
                        table_v)

        def scale_body(k, carry):
            for j in range(8):
                sl = pl.ds(k * 8 * _LANES + j * _LANES, _LANES)
                idx_v[sl] = idx_v[sl] * feat
            return carry

        lax.fori_loop(0, per_w // (8 * _LANES), scale_body, 0)

        rows = (rows0, rows1)
        ss = (ss0, ss1)
        lane_iota = lax.iota(jnp.int32, _LANES)
        cg_offs = [lane_iota + cg * _LANES for cg in range(feat // _LANES)]

        def build_chunk(i, bf):
            buf = rows[bf]
            i0 = i * _CHUNK

            def row_body(j, carry):
                kb = plsc.load_gather(
                    idx_v, [jnp.full((_LANES,), i0 + j, jnp.int32)])
                dst = j * feat
                for cg in range(feat // _LANES):
                    v = plsc.load_gather(table_v, [kb + cg_offs[cg]])
                    buf[pl.ds(dst + cg * _LANES, _LANES)] = v
                return carry

            lax.fori_loop(0, _CHUNK, row_body, 0)

        def store_desc(i, bf):
            return pltpu.make_async_copy(
                rows[bf],
                out_hbm.at[pl.ds((base + i * _CHUNK) * feat, chunk_elems)],
                ss[bf])

        build_chunk(0, 0)

        def loop_body(g, carry):
            i = 2 * g

            @pl.when(g >= 1)
            def _():
                store_desc(i - 1, 1).wait()

            store_desc(i, 0).start()
            build_chunk(i + 1, 1)
            store_desc(i, 0).wait()
            store_desc(i + 1, 1).start()

            @pl.when(g < n_chunks // 2 - 1)
            def _():
                build_chunk(i + 2, 0)

            return carry

        lax.fori_loop(0, n_chunks // 2, loop_body, 0)
        store_desc(n_chunks - 1, 1).wait()

    return gather_kernel


def kernel(lcas, features):
    batch, length, feat = features.shape
    trows = -(-(length + 1) // 8) * 8
    table = jnp.concatenate(
        [jnp.zeros((batch, 1, feat), features.dtype), features,
         jnp.zeros((batch, trows - length - 1, feat), features.dtype)],
        axis=1,
    ).reshape(batch * trows * feat)
    idx = lcas.astype(jnp.int32).reshape(-1)
    total = batch * length * length
    out = _make_gather(total, feat, length * length, trows)(idx, table)
    return out.reshape(batch, length, length, feat)


# speedup vs baseline: 1.6725x; 1.6725x over previous
import functools

import jax
import jax.numpy as jnp
from jax import lax
from jax.experimental import pallas as pl
from jax.experimental.pallas import tpu as pltpu
from jax.experimental.pallas import tpu_sc as plsc

_LANES = 16
_CH = 64
_R = 48
_NBUF = 3


@functools.lru_cache(maxsize=None)
def _make_gather(total_rows, feat, rows_per_batch, table_rows_per_batch):
    info = plsc.get_sparse_core_info()
    nc, ns = info.num_cores, info.num_subcores
    nw = nc * ns
    per_w = total_rows // nw
    n_chunks = per_w // _CH
    n_local_rows = (n_chunks - _R) * _CH
    assert _R % _NBUF == 0 and (_R * _CH) % (8 * _LANES) == 0
    assert 0 < n_local_rows < per_w
    assert rows_per_batch % per_w == 0
    mesh = plsc.VectorSubcoreMesh(core_axis_name="c", subcore_axis_name="s")

    @functools.partial(
        pl.kernel,
        mesh=mesh,
        compiler_params=pltpu.CompilerParams(use_tc_tiling_on_sc=False,
                                             needs_layout_passes=False),
        out_type=jax.ShapeDtypeStruct((total_rows, feat), jnp.float32),
        scratch_types=[
            pltpu.VMEM((per_w,), jnp.int32),
            pltpu.VMEM((table_rows_per_batch, feat), jnp.float32),
            pltpu.VMEM((_NBUF, _CH, feat), jnp.float32),
            pltpu.VMEM((2 * _CH, feat), jnp.float32),
            pltpu.SemaphoreType.DMA,
            pltpu.SemaphoreType.DMA,
            pltpu.SemaphoreType.DMA,
            pltpu.SemaphoreType.DMA,
            pltpu.SemaphoreType.DMA,
            pltpu.SemaphoreType.DMA,
            pltpu.SemaphoreType.DMA,
        ],
    )
    def gather_kernel(idx_hbm, table_hbm, out_hbm, idx_v, table_v, rbuf,
                      lbuf, sg0, sg1, sg2, ss0, ss1, ss2, sl):
        wid = lax.axis_index("s") * nc + lax.axis_index("c")
        base = wid * per_w
        b = base // rows_per_batch
        off = b * table_rows_per_batch

        pltpu.sync_copy(idx_hbm.at[pl.ds(base, per_w)], idx_v)
        pltpu.sync_copy(
            table_hbm.at[pl.ds(b * table_rows_per_batch,
                               table_rows_per_batch)], table_v)

        def adj_body(k, carry):
            for j in range(8):
                sl_ = pl.ds(k * 8 * _LANES + j * _LANES, _LANES)
                idx_v[sl_] = idx_v[sl_] + off
            return carry

        lax.fori_loop(0, (_R * _CH) // (8 * _LANES), adj_body, 0)

        sg = (sg0, sg1, sg2)
        ss = (ss0, ss1, ss2)
        lane_iota = lax.iota(jnp.int32, _LANES)
        cg_offs = [lane_iota + cg * _LANES for cg in range(feat // _LANES)]

        def gather_desc(i, bf):
            return pltpu.make_async_copy(
                table_hbm.at[idx_v.at[pl.ds(i * _CH, _CH)]],
                rbuf.at[bf], sg[bf])

        def store_desc(i, bf):
            return pltpu.make_async_copy(
                rbuf.at[bf], out_hbm.at[pl.ds(base + i * _CH, _CH)], ss[bf])

        def lstore_desc(h):
            return pltpu.make_async_copy(
                lbuf.at[pl.ds((h % 2) * _CH, _CH)],
                out_hbm.at[pl.ds(base + (_R + h) * _CH, _CH)], sl)

        def build_row(lr, carry):
            slot = lr % (2 * _CH)
            kb = plsc.load_gather(
                idx_v, [jnp.full((_LANES,), _R * _CH + lr, jnp.int32)])
            for cg in range(feat // _LANES):
                v = plsc.load_gather(table_v, [kb, cg_offs[cg]])
                lbuf[slot, pl.ds(cg * _LANES, _LANES)] = v
            return carry

        gather_desc(0, 0).start()
        gather_desc(1, 1).start()

        def loop_body(g, carry):
            for bf in range(_NBUF):
                i = _NBUF * g + bf
                gather_desc(i, bf).wait()
                if bf == 0:
                    @pl.when(g >= 1)
                    def _():
                        store_desc(i - 1, _NBUF - 1).wait()
                else:
                    store_desc(i - 1, bf - 1).wait()

                @pl.when(i + 2 < _R)
                def _():
                    gather_desc(i + 2, (bf + 2) % _NBUF).start()

                store_desc(i, bf).start()

                lr_lo = (i * n_local_rows) // _R
                lr_hi = ((i + 1) * n_local_rows) // _R
                lax.fori_loop(lr_lo, lr_hi, build_row, 0)
                h_lo = lr_lo // _CH
                h_hi = lr_hi // _CH

                @pl.when(h_hi > h_lo)
                def _():
                    @pl.when(h_lo >= 1)
                    def _():
                        lstore_desc(h_lo - 1).wait()

                    lstore_desc(h_lo).start()

            return carry

        lax.fori_loop(0, _R // _NBUF, loop_body, 0)
        store_desc(_R - 1, (_R - 1) % _NBUF).wait()
        lstore_desc(n_chunks - _R - 1).wait()

    return gather_kernel


def kernel(lcas, features):
    batch, length, feat = features.shape
    trows = -(-(length + 1) // 8) * 8
    table = jnp.concatenate(
        [jnp.zeros((batch, 1, feat), features.dtype), features,
         jnp.zeros((batch, trows - length - 1, feat), features.dtype)],
        axis=1,
    ).reshape(batch * trows, feat)
    idx = lcas.astype(jnp.int32).reshape(-1)
    total = batch * length * length
    out = _make_gather(total, feat, length * length, trows)(idx, table)
    return out.reshape(batch, length, length, feat)


# speedup vs baseline: 3.2869x; 1.9653x over previous
import functools

import jax
import jax.numpy as jnp
from jax import lax
from jax.experimental import pallas as pl
from jax.experimental.pallas import tpu as pltpu
from jax.experimental.pallas import tpu_sc as plsc

_LANES = 16
_CH = 64
_NBUF = 4


@functools.lru_cache(maxsize=None)
def _make_gather(total_rows, feat, rows_per_batch, table_rows_per_batch):
    info = plsc.get_sparse_core_info()
    nc, ns = info.num_cores, info.num_subcores
    nw = nc * ns
    per_w = total_rows // nw
    n_chunks = per_w // _CH
    assert n_chunks % _NBUF == 0
    assert rows_per_batch % per_w == 0
    mesh = plsc.VectorSubcoreMesh(core_axis_name="c", subcore_axis_name="s")

    @functools.partial(
        pl.kernel,
        mesh=mesh,
        out_type=jax.ShapeDtypeStruct((total_rows, feat), jnp.float32),
        scratch_types=[
            pltpu.VMEM((per_w,), jnp.int32),
            pltpu.VMEM((_NBUF, _CH, feat), jnp.float32),
            pltpu.SemaphoreType.DMA,
            pltpu.SemaphoreType.DMA,
            pltpu.SemaphoreType.DMA,
            pltpu.SemaphoreType.DMA,
            pltpu.SemaphoreType.DMA,
            pltpu.SemaphoreType.DMA,
            pltpu.SemaphoreType.DMA,
            pltpu.SemaphoreType.DMA,
        ],
    )
    def gather_kernel(idx_hbm, table_hbm, out_hbm, idx_v, rbuf,
                      sg0, sg1, sg2, sg3, ss0, ss1, ss2, ss3):
        wid = lax.axis_index("s") * nc + lax.axis_index("c")
        base = wid * per_w
        off = (base // rows_per_batch) * table_rows_per_batch

        pltpu.sync_copy(idx_hbm.at[pl.ds(base, per_w)], idx_v)

        def adj_body(k, carry):
            for j in range(8):
                sl = pl.ds(k * 8 * _LANES + j * _LANES, _LANES)
                idx_v[sl] = idx_v[sl] + off
            return carry

        lax.fori_loop(0, per_w // (8 * _LANES), adj_body, 0)

        sg = (sg0, sg1, sg2, sg3)
        ss = (ss0, ss1, ss2, ss3)

        def gather_desc(i, bf):
            return pltpu.make_async_copy(
                table_hbm.at[idx_v.at[pl.ds(i * _CH, _CH)]],
                rbuf.at[bf], sg[bf])

        def store_desc(i, bf):
            return pltpu.make_async_copy(
                rbuf.at[bf], out_hbm.at[pl.ds(base + i * _CH, _CH)], ss[bf])

        gather_desc(0, 0).start()
        gather_desc(1, 1).start()

        def loop_body(g, carry):
            for bf in range(_NBUF):
                i = _NBUF * g + bf
                gather_desc(i, bf).wait()
                nb = (bf + 2) % _NBUF
                if bf < 2:
                    @pl.when(g >= 1)
                    def _():
                        store_desc(i - 2, nb).wait()
                else:
                    store_desc(i - 2, nb).wait()

                @pl.when(i + 2 < n_chunks)
                def _():
                    gather_desc(i + 2, nb).start()

                store_desc(i, bf).start()
            return carry

        lax.fori_loop(0, n_chunks // _NBUF, loop_body, 0)
        store_desc(n_chunks - 2, (n_chunks - 2) % _NBUF).wait()
        store_desc(n_chunks - 1, (n_chunks - 1) % _NBUF).wait()

    return gather_kernel


def kernel(lcas, features):
    batch, length, feat = features.shape
    table = jnp.concatenate(
        [jnp.zeros((batch, 1, feat), features.dtype), features], axis=1
    ).reshape(batch * (length + 1), feat)
    idx = lcas.astype(jnp.int32).reshape(-1)
    total = batch * length * length
    out = _make_gather(total, feat, length * length, length + 1)(idx, table)
    return out.reshape(batch, length, length, feat)
